# baseline (speedup 1.0000x reference)
import functools

import jax
import jax.numpy as jnp
import numpy as np
from jax import lax
from jax.experimental import pallas as pl
from jax.experimental.pallas import tpu as pltpu
from jax.experimental.pallas import tpu_sc as plsc

F32 = jnp.float32

_hdot = functools.partial(jnp.dot, precision=lax.Precision.HIGHEST,
                          preferred_element_type=F32)
_fdot = functools.partial(jnp.dot, precision=lax.Precision.DEFAULT,
                          preferred_element_type=F32)


BF16 = jnp.bfloat16


def _split2(v):
    hi = v.astype(BF16)
    lo = (v - hi.astype(F32)).astype(BF16)
    return hi, lo


def _bdot(a, b):
    return jnp.dot(a, b, preferred_element_type=F32)


def _dot3(a1, a2, bhi, blo):
    return (_bdot(a1, bhi) + _bdot(a1, blo)) + _bdot(a2, bhi)


def _route_body(x_ref, lmat_ref, pmat_ref, w1hi_ref, w1lo_ref, b1_ref,
                w2hi_ref, w2lo_ref, b2_ref,
                xg_ref, top2_ref, m_ref, idx1_ref, idx2_ref):
    xb = x_ref[0]
    x1, x2 = _split2(xb)
    lm = lmat_ref[...]
    z = _bdot(lm, x1) + _bdot(lm, x2)
    z1, z2 = _split2(z)
    pm = pmat_ref[...]
    pooled = (_bdot(z1, pm) + _bdot(z2, pm)) * (1.0 / 196.0)

    p1, p2 = _split2(pooled)
    accs = [None] * 4
    for a in range(48):
        d = _dot3(p1[a:a + 1, :], p2[a:a + 1, :], w1hi_ref[a], w1lo_ref[a])
        g = a % 4
        accs[g] = d if accs[g] is None else accs[g] + d
    acc = b1_ref[...] + ((accs[0] + accs[1]) + (accs[2] + accs[3]))
    f = jax.nn.relu(acc)
    f1, f2 = _split2(f)
    lg = _dot3(f1, f2, w2hi_ref[...], w2lo_ref[...]) + b2_ref[...]

    io64 = lax.broadcasted_iota(jnp.int32, (1, 64), 1)
    m1 = jnp.max(lg, axis=1, keepdims=True)
    i1 = jnp.min(jnp.where(lg == m1, io64, 64), axis=1, keepdims=True)
    lg2 = jnp.where(io64 == i1, F32(-1e30), lg)
    m2 = jnp.max(lg2, axis=1, keepdims=True)
    i2 = jnp.min(jnp.where(lg2 == m2, io64, 64), axis=1, keepdims=True)
    io128 = lax.broadcasted_iota(jnp.int32, (1, 128), 1)
    top2_ref[0] = jnp.where(io128 == 0,
                            jnp.broadcast_to(i1, (1, 128)),
                            jnp.broadcast_to(i2, (1, 128)))

    ioa = lax.broadcasted_iota(jnp.int32, (1, 1024), 1)
    e_sela = jnp.where(ioa < 512,
                       jnp.broadcast_to(i1, (1, 1024)),
                       jnp.broadcast_to(i2, (1, 1024)))
    idx1_ref[0] = e_sela * 512 + (ioa & 511)
    iob = lax.broadcasted_iota(jnp.int32, (1, 512), 1)
    e_selb = jnp.where(iob < 256,
                       jnp.broadcast_to(i1, (1, 512)),
                       jnp.broadcast_to(i2, (1, 512)))
    idx2_ref[0] = e_selb * 256 + (iob & 255)

    parts = [f]
    for k in range(4):
        q = f[:, k * 128:(k + 1) * 128]
        parts.append(jnp.pad(q, ((0, 0), (0, 384))))
    xg = jnp.concatenate(parts, axis=0)
    xg_ref[0] = xg

    diff = xg[:, None, :] - xg[None, :, :]
    d2 = jnp.sum(diff * diff, axis=-1)
    io5c = lax.broadcasted_iota(jnp.int32, (5, 5), 1)
    io5r = lax.broadcasted_iota(jnp.int32, (5, 5), 0)
    mx = jnp.max(d2, axis=1, keepdims=True)
    excl = jnp.max(jnp.where(d2 == mx, io5c, -1), axis=1, keepdims=True)
    m_ref[0] = ((io5c != excl).astype(F32) + (io5c == io5r).astype(F32))


def _route(x3, lmat, pmat, w1, b1, w2, b2):
    bn = x3.shape[0]
    w1b = w1.reshape(48, 16, 512)
    w1hi = w1b.astype(BF16)
    w1lo = (w1b - w1hi.astype(F32)).astype(BF16)
    w2hi = w2.astype(BF16)
    w2lo = (w2 - w2hi.astype(F32)).astype(BF16)
    return pl.pallas_call(
        _route_body,
        grid=(bn,),
        in_specs=[
            pl.BlockSpec((1, 672, 224), lambda b: (b, 0, 0)),
            pl.BlockSpec((48, 672), lambda b: (0, 0)),
            pl.BlockSpec((224, 16), lambda b: (0, 0)),
            pl.BlockSpec((48, 16, 512), lambda b: (0, 0, 0)),
            pl.BlockSpec((48, 16, 512), lambda b: (0, 0, 0)),
            pl.BlockSpec((1, 512), lambda b: (0, 0)),
            pl.BlockSpec((512, 64), lambda b: (0, 0)),
            pl.BlockSpec((512, 64), lambda b: (0, 0)),
            pl.BlockSpec((1, 64), lambda b: (0, 0)),
        ],
        out_specs=[
            pl.BlockSpec((1, 5, 512), lambda b: (b, 0, 0)),
            pl.BlockSpec((1, 1, 128), lambda b: (b, 0, 0)),
            pl.BlockSpec((1, 5, 5), lambda b: (b, 0, 0)),
            pl.BlockSpec((1, 1, 1024), lambda b: (b, 0, 0)),
            pl.BlockSpec((1, 1, 512), lambda b: (b, 0, 0)),
        ],
        out_shape=[
            jax.ShapeDtypeStruct((bn, 5, 512), F32),
            jax.ShapeDtypeStruct((bn, 1, 128), jnp.int32),
            jax.ShapeDtypeStruct((bn, 5, 5), F32),
            jax.ShapeDtypeStruct((bn, 1, 1024), jnp.int32),
            jax.ShapeDtypeStruct((bn, 1, 512), jnp.int32),
        ],
    )(x3, lmat.astype(BF16), pmat.astype(BF16), w1hi, w1lo,
      b1.reshape(1, 512), w2hi, w2lo, b2.reshape(1, 64))


def _sc_gather(idx1_flat, idx2_flat, top2_flat, w1_view, w2_view, b1, b2):
    mesh = plsc.VectorSubcoreMesh(core_axis_name="c", subcore_axis_name="s")

    @functools.partial(
        pl.kernel,
        out_type=(
            jax.ShapeDtypeStruct((8192, 256), F32),
            jax.ShapeDtypeStruct((4096, 128), F32),
            jax.ShapeDtypeStruct((16, 256), F32),
            jax.ShapeDtypeStruct((16, 128), F32),
        ),
        mesh=mesh,
        scratch_types=(
            pltpu.VMEM((128,), jnp.int32),
            pltpu.VMEM((128,), jnp.int32),
            pltpu.VMEM((128,), jnp.int32),
            pltpu.VMEM((16,), jnp.int32),
            pltpu.VMEM((256, 256), F32),
            pltpu.VMEM((128, 128), F32),
            pltpu.VMEM((16, 256), F32),
            pltpu.VMEM((16, 128), F32),
            pltpu.SemaphoreType.DMA,
            pltpu.SemaphoreType.DMA,
            pltpu.SemaphoreType.DMA,
        ),
    )
    def gather_k(idx1_hbm, idx2_hbm, top2_hbm, w1_hbm, w2_hbm, b1_hbm, b2_hbm,
                 o_w1, o_w2, o_b1, o_b2,
                 ia_v, ib_v, ic_v, e_v, rows1_v, rows2_v, b1_v, b2_v,
                 sem, semb, semo):
        wid = lax.axis_index("s") * 2 + lax.axis_index("c")
        pltpu.sync_copy(idx1_hbm.at[pl.ds(wid * 256, 128)], ia_v)
        pltpu.sync_copy(idx1_hbm.at[pl.ds(wid * 256 + 128, 128)], ib_v)
        pltpu.sync_copy(idx2_hbm.at[pl.ds(wid * 128, 128)], ic_v)
        c1 = pltpu.async_copy(w1_hbm.at[ia_v], rows1_v.at[pl.ds(0, 128)], sem)
        c2 = pltpu.async_copy(w1_hbm.at[ib_v], rows1_v.at[pl.ds(128, 128)], sem)
        c3 = pltpu.async_copy(w2_hbm.at[ic_v], rows2_v, sem)
        c1.wait()
        o1 = pltpu.async_copy(rows1_v.at[pl.ds(0, 128)],
                              o_w1.at[pl.ds(wid * 256, 128)], semo)
        c2.wait()
        o2 = pltpu.async_copy(rows1_v.at[pl.ds(128, 128)],
                              o_w1.at[pl.ds(wid * 256 + 128, 128)], semo)
        c3.wait()
        o3 = pltpu.async_copy(rows2_v, o_w2.at[pl.ds(wid * 128, 128)], semo)
        o1.wait()
        o2.wait()
        o3.wait()

        @pl.when(wid == 0)
        def _():
            pltpu.sync_copy(top2_hbm, e_v)
            pltpu.async_copy(b1_hbm.at[e_v], b1_v, semb).wait()
            pltpu.sync_copy(b1_v, o_b1)

        @pl.when(wid == 1)
        def _():
            pltpu.sync_copy(top2_hbm, e_v)
            pltpu.async_copy(b2_hbm.at[e_v], b2_v, semb).wait()
            pltpu.sync_copy(b2_v, o_b2)

    return gather_k(idx1_flat, idx2_flat, top2_flat, w1_view, w2_view, b1, b2)


def _experts_body(xg_ref, m_ref, w1_ref, w2_ref, b1_ref, b2_ref,
                  fw_ref, fb_ref, o_ref, acc_ref):
    t = pl.program_id(0)
    xg = xg_ref[0]
    mm = m_ref[0]
    xw = _fdot(xg, w1_ref[0])
    h = jax.nn.relu(_fdot(mm, xw) * 0.2 + b1_ref[pl.ds(t, 1), :])
    h2 = _fdot(mm, _fdot(h, w2_ref[0])) * 0.2 + b2_ref[pl.ds(t, 1), :]
    acc_ref[pl.ds(t, 1), :] = jnp.mean(h2, axis=0, keepdims=True)

    @pl.when(t == 15)
    def _():
        comb = acc_ref[...]
        fin = _fdot(comb, fw_ref[...])
        io5c = lax.broadcasted_iota(jnp.int32, (5, 5), 1)
        io5r = lax.broadcasted_iota(jnp.int32, (5, 5), 0)
        c7 = mm - (io5c == io5r).astype(F32)
        c7p = jnp.pad(c7, ((0, 11), (0, 11)))
        r16 = lax.broadcasted_iota(jnp.int32, (16, 16), 0)
        c16 = lax.broadcasted_iota(jnp.int32, (16, 16), 1)
        diag = jnp.where(r16 == c16,
                         jnp.where(r16 < 5, F32(0.2), F32(1.0)), F32(0.0))
        mf = diag + c7p * 0.2
        fin2 = _fdot(mf, fin) + fb_ref[...]
        r8 = lax.broadcasted_iota(jnp.int32, (8, 16), 0)
        c8 = lax.broadcasted_iota(jnp.int32, (8, 16), 1)
        pairmean = ((c8 == 2 * r8) | (c8 == 2 * r8 + 1)).astype(F32)
        o_ref[...] = _fdot(pairmean, fin2) * 0.5


def _experts(xg, m, w1g, w2g, b1g, b2g, fw, fb):
    return pl.pallas_call(
        _experts_body,
        grid=(16,),
        in_specs=[
            pl.BlockSpec((1, 5, 512), lambda t: (t // 2, 0, 0)),
            pl.BlockSpec((1, 5, 5), lambda t: (t // 2, 0, 0)),
            pl.BlockSpec((1, 512, 256), lambda t: (t, 0, 0)),
            pl.BlockSpec((1, 256, 128), lambda t: (t, 0, 0)),
            pl.BlockSpec((16, 256), lambda t: (0, 0)),
            pl.BlockSpec((16, 128), lambda t: (0, 0)),
            pl.BlockSpec((128, 64), lambda t: (0, 0)),
            pl.BlockSpec((1, 64), lambda t: (0, 0)),
        ],
        out_specs=pl.BlockSpec((8, 64), lambda t: (0, 0)),
        out_shape=jax.ShapeDtypeStruct((8, 64), F32),
        scratch_shapes=[pltpu.VMEM((16, 128), F32)],
    )(xg, m, w1g, w2g, b1g, b2g, fw, fb)


def _make_pool_consts():
    lmat = np.zeros((48, 672), dtype=np.float32)
    for a in range(48):
        ch, i = divmod(a, 16)
        lmat[a, ch * 224 + i * 14:(ch * 224 + (i + 1) * 14)] = 1.0
    pmat = np.zeros((224, 16), dtype=np.float32)
    for rr in range(224):
        pmat[rr, rr // 14] = 1.0
    return jnp.asarray(lmat), jnp.asarray(pmat)


def kernel(x, yolo_W1, yolo_b1, yolo_W2, yolo_b2,
           gnn_W1, gnn_b1, gnn_W2, gnn_b2, final_W, final_b):
    bn = x.shape[0]
    lmat, pmat = _make_pool_consts()
    xg, top2_3d, m, idx1_3d, idx2_3d = _route(
        x.reshape(bn, 672, 224), lmat, pmat,
        yolo_W1, yolo_b1, yolo_W2, yolo_b2)
    top2_flat = top2_3d[:, 0, :2].reshape(2 * bn).astype(jnp.int32)

    o_w1, o_w2, b1g, b2g = _sc_gather(
        idx1_3d.reshape(1024 * bn), idx2_3d.reshape(512 * bn), top2_flat,
        gnn_W1.reshape(64 * 512, 256),
        gnn_W2.reshape(64 * 256, 128),
        gnn_b1, gnn_b2)
    w1g = o_w1.reshape(16, 512, 256)
    w2g = o_w2.reshape(16, 256, 128)

    return _experts(xg, m, w1g, w2g, b1g, b2g,
                    final_W, final_b.reshape(1, 64))


# speedup vs baseline: 1.0842x; 1.0842x over previous
import functools

import jax
import jax.numpy as jnp
import numpy as np
from jax import lax
from jax.experimental import pallas as pl
from jax.experimental.pallas import tpu as pltpu
from jax.experimental.pallas import tpu_sc as plsc

F32 = jnp.float32

_hdot = functools.partial(jnp.dot, precision=lax.Precision.HIGHEST,
                          preferred_element_type=F32)
_fdot = functools.partial(jnp.dot, precision=lax.Precision.DEFAULT,
                          preferred_element_type=F32)


BF16 = jnp.bfloat16


def _split2(v):
    hi = v.astype(BF16)
    lo = (v - hi.astype(F32)).astype(BF16)
    return hi, lo


def _bdot(a, b):
    return jnp.dot(a, b, preferred_element_type=F32)


def _dot3(a1, a2, bhi, blo):
    return (_bdot(a1, bhi) + _bdot(a1, blo)) + _bdot(a2, bhi)


def _route_body(x_ref, lmat_ref, pmat_ref, w1hi_ref, w1lo_ref, b1_ref,
                w2hi_ref, w2lo_ref, b2_ref,
                xg_ref, top2_ref, m_ref, idx2_ref):
    xb = x_ref[0]
    x1, x2 = _split2(xb)
    lm = lmat_ref[...]
    z = _bdot(lm, x1) + _bdot(lm, x2)
    z1, z2 = _split2(z)
    pm = pmat_ref[...]
    pooled = (_bdot(z1, pm) + _bdot(z2, pm)) * (1.0 / 196.0)

    p1, p2 = _split2(pooled)
    accs = [None] * 4
    for a in range(48):
        d = _dot3(p1[a:a + 1, :], p2[a:a + 1, :], w1hi_ref[a], w1lo_ref[a])
        g = a % 4
        accs[g] = d if accs[g] is None else accs[g] + d
    acc = b1_ref[...] + ((accs[0] + accs[1]) + (accs[2] + accs[3]))
    f = jax.nn.relu(acc)
    f1, f2 = _split2(f)
    lg = _dot3(f1, f2, w2hi_ref[...], w2lo_ref[...]) + b2_ref[...]

    io64 = lax.broadcasted_iota(jnp.int32, (1, 64), 1)
    m1 = jnp.max(lg, axis=1, keepdims=True)
    i1 = jnp.min(jnp.where(lg == m1, io64, 64), axis=1, keepdims=True)
    lg2 = jnp.where(io64 == i1, F32(-1e30), lg)
    m2 = jnp.max(lg2, axis=1, keepdims=True)
    i2 = jnp.min(jnp.where(lg2 == m2, io64, 64), axis=1, keepdims=True)
    io128 = lax.broadcasted_iota(jnp.int32, (1, 128), 1)
    top2_ref[0] = jnp.where(io128 == 0,
                            jnp.broadcast_to(i1, (1, 128)),
                            jnp.broadcast_to(i2, (1, 128)))

    iob = lax.broadcasted_iota(jnp.int32, (1, 512), 1)
    e_selb = jnp.where(iob < 256,
                       jnp.broadcast_to(i1, (1, 512)),
                       jnp.broadcast_to(i2, (1, 512)))
    idx2_ref[0] = e_selb * 256 + (iob & 255)

    parts = [f]
    for k in range(4):
        q = f[:, k * 128:(k + 1) * 128]
        parts.append(jnp.pad(q, ((0, 0), (0, 384))))
    xg = jnp.concatenate(parts, axis=0)
    xg_ref[0] = xg

    diff = xg[:, None, :] - xg[None, :, :]
    d2 = jnp.sum(diff * diff, axis=-1)
    io5c = lax.broadcasted_iota(jnp.int32, (5, 5), 1)
    io5r = lax.broadcasted_iota(jnp.int32, (5, 5), 0)
    mx = jnp.max(d2, axis=1, keepdims=True)
    excl = jnp.max(jnp.where(d2 == mx, io5c, -1), axis=1, keepdims=True)
    m_ref[0] = ((io5c != excl).astype(F32) + (io5c == io5r).astype(F32))


def _route(x3, lmat, pmat, w1, b1, w2, b2):
    bn = x3.shape[0]
    w1b = w1.reshape(48, 16, 512)
    w1hi = w1b.astype(BF16)
    w1lo = (w1b - w1hi.astype(F32)).astype(BF16)
    w2hi = w2.astype(BF16)
    w2lo = (w2 - w2hi.astype(F32)).astype(BF16)
    return pl.pallas_call(
        _route_body,
        grid=(bn,),
        in_specs=[
            pl.BlockSpec((1, 672, 224), lambda b: (b, 0, 0)),
            pl.BlockSpec((48, 672), lambda b: (0, 0)),
            pl.BlockSpec((224, 16), lambda b: (0, 0)),
            pl.BlockSpec((48, 16, 512), lambda b: (0, 0, 0)),
            pl.BlockSpec((48, 16, 512), lambda b: (0, 0, 0)),
            pl.BlockSpec((1, 512), lambda b: (0, 0)),
            pl.BlockSpec((512, 64), lambda b: (0, 0)),
            pl.BlockSpec((512, 64), lambda b: (0, 0)),
            pl.BlockSpec((1, 64), lambda b: (0, 0)),
        ],
        out_specs=[
            pl.BlockSpec((1, 5, 512), lambda b: (b, 0, 0)),
            pl.BlockSpec((1, 1, 128), lambda b: (b, 0, 0)),
            pl.BlockSpec((1, 5, 5), lambda b: (b, 0, 0)),
            pl.BlockSpec((1, 1, 512), lambda b: (b, 0, 0)),
        ],
        out_shape=[
            jax.ShapeDtypeStruct((bn, 5, 512), F32),
            jax.ShapeDtypeStruct((bn, 1, 128), jnp.int32),
            jax.ShapeDtypeStruct((bn, 5, 5), F32),
            jax.ShapeDtypeStruct((bn, 1, 512), jnp.int32),
        ],
    )(x3, lmat.astype(BF16), pmat.astype(BF16), w1hi, w1lo,
      b1.reshape(1, 512), w2hi, w2lo, b2.reshape(1, 64))


def _sc_gather(idx2_flat, top2_flat, w2_view, b1, b2):
    mesh = plsc.VectorSubcoreMesh(core_axis_name="c", subcore_axis_name="s")

    @functools.partial(
        pl.kernel,
        out_type=(
            jax.ShapeDtypeStruct((4096, 128), F32),
            jax.ShapeDtypeStruct((16, 256), F32),
            jax.ShapeDtypeStruct((16, 128), F32),
        ),
        mesh=mesh,
        scratch_types=(
            pltpu.VMEM((128,), jnp.int32),
            pltpu.VMEM((16,), jnp.int32),
            pltpu.VMEM((128, 128), F32),
            pltpu.VMEM((16, 256), F32),
            pltpu.VMEM((16, 128), F32),
            pltpu.SemaphoreType.DMA,
            pltpu.SemaphoreType.DMA,
            pltpu.SemaphoreType.DMA,
        ),
    )
    def gather_k(idx2_hbm, top2_hbm, w2_hbm, b1_hbm, b2_hbm,
                 o_w2, o_b1, o_b2,
                 ic_v, e_v, rows2_v, b1_v, b2_v,
                 sem, semb, semo):
        wid = lax.axis_index("s") * 2 + lax.axis_index("c")
        pltpu.sync_copy(idx2_hbm.at[pl.ds(wid * 128, 128)], ic_v)
        c3 = pltpu.async_copy(w2_hbm.at[ic_v], rows2_v, sem)
        c3.wait()
        o3 = pltpu.async_copy(rows2_v, o_w2.at[pl.ds(wid * 128, 128)], semo)
        o3.wait()

        @pl.when(wid == 0)
        def _():
            pltpu.sync_copy(top2_hbm, e_v)
            pltpu.async_copy(b1_hbm.at[e_v], b1_v, semb).wait()
            pltpu.sync_copy(b1_v, o_b1)

        @pl.when(wid == 1)
        def _():
            pltpu.sync_copy(top2_hbm, e_v)
            pltpu.async_copy(b2_hbm.at[e_v], b2_v, semb).wait()
            pltpu.sync_copy(b2_v, o_b2)

    return gather_k(idx2_flat, top2_flat, w2_view, b1, b2)


def _experts_body(e_ref, xg_ref, m_ref, w1_ref, w2_ref, b1_ref, b2_ref,
                  fw_ref, fb_ref, o_ref, acc_ref):
    t = pl.program_id(0)
    xg = xg_ref[0]
    mm = m_ref[0]
    xw = _fdot(xg, w1_ref[0])
    h = jax.nn.relu(_fdot(mm, xw) * 0.2 + b1_ref[pl.ds(t, 1), :])
    h2 = _fdot(mm, _fdot(h, w2_ref[0])) * 0.2 + b2_ref[pl.ds(t, 1), :]
    acc_ref[pl.ds(t, 1), :] = jnp.mean(h2, axis=0, keepdims=True)

    @pl.when(t == 15)
    def _():
        comb = acc_ref[...]
        fin = _fdot(comb, fw_ref[...])
        io5c = lax.broadcasted_iota(jnp.int32, (5, 5), 1)
        io5r = lax.broadcasted_iota(jnp.int32, (5, 5), 0)
        c7 = mm - (io5c == io5r).astype(F32)
        c7p = jnp.pad(c7, ((0, 11), (0, 11)))
        r16 = lax.broadcasted_iota(jnp.int32, (16, 16), 0)
        c16 = lax.broadcasted_iota(jnp.int32, (16, 16), 1)
        diag = jnp.where(r16 == c16,
                         jnp.where(r16 < 5, F32(0.2), F32(1.0)), F32(0.0))
        mf = diag + c7p * 0.2
        fin2 = _fdot(mf, fin) + fb_ref[...]
        r8 = lax.broadcasted_iota(jnp.int32, (8, 16), 0)
        c8 = lax.broadcasted_iota(jnp.int32, (8, 16), 1)
        pairmean = ((c8 == 2 * r8) | (c8 == 2 * r8 + 1)).astype(F32)
        o_ref[...] = _fdot(pairmean, fin2) * 0.5


def _experts(top2i, xg, m, w1_full, w2g, b1g, b2g, fw, fb):
    grid_spec = pltpu.PrefetchScalarGridSpec(
        num_scalar_prefetch=1,
        grid=(16,),
        in_specs=[
            pl.BlockSpec((1, 5, 512), lambda t, e: (t // 2, 0, 0)),
            pl.BlockSpec((1, 5, 5), lambda t, e: (t // 2, 0, 0)),
            pl.BlockSpec((1, 512, 256), lambda t, e: (e[t], 0, 0)),
            pl.BlockSpec((1, 256, 128), lambda t, e: (t, 0, 0)),
            pl.BlockSpec((16, 256), lambda t, e: (0, 0)),
            pl.BlockSpec((16, 128), lambda t, e: (0, 0)),
            pl.BlockSpec((128, 64), lambda t, e: (0, 0)),
            pl.BlockSpec((1, 64), lambda t, e: (0, 0)),
        ],
        out_specs=pl.BlockSpec((8, 64), lambda t, e: (0, 0)),
        scratch_shapes=[pltpu.VMEM((16, 128), F32)],
    )
    return pl.pallas_call(
        _experts_body,
        grid_spec=grid_spec,
        out_shape=jax.ShapeDtypeStruct((8, 64), F32),
    )(top2i, xg, m, w1_full, w2g, b1g, b2g, fw, fb)


def _make_pool_consts():
    lmat = np.zeros((48, 672), dtype=np.float32)
    for a in range(48):
        ch, i = divmod(a, 16)
        lmat[a, ch * 224 + i * 14:(ch * 224 + (i + 1) * 14)] = 1.0
    pmat = np.zeros((224, 16), dtype=np.float32)
    for rr in range(224):
        pmat[rr, rr // 14] = 1.0
    return jnp.asarray(lmat), jnp.asarray(pmat)


def kernel(x, yolo_W1, yolo_b1, yolo_W2, yolo_b2,
           gnn_W1, gnn_b1, gnn_W2, gnn_b2, final_W, final_b):
    bn = x.shape[0]
    lmat, pmat = _make_pool_consts()
    xg, top2_3d, m, idx2_3d = _route(
        x.reshape(bn, 672, 224), lmat, pmat,
        yolo_W1, yolo_b1, yolo_W2, yolo_b2)
    top2_flat = top2_3d[:, 0, :2].reshape(2 * bn).astype(jnp.int32)

    o_w2, b1g, b2g = _sc_gather(
        idx2_3d.reshape(512 * bn), top2_flat,
        gnn_W2.reshape(64 * 256, 128),
        gnn_b1, gnn_b2)
    w2g = o_w2.reshape(16, 256, 128)

    return _experts(top2_flat, xg, m, gnn_W1, w2g, b1g, b2g,
                    final_W, final_b.reshape(1, 64))


# speedup vs baseline: 1.1462x; 1.0572x over previous
import functools

import jax
import jax.numpy as jnp
import numpy as np
from jax import lax
from jax.experimental import pallas as pl
from jax.experimental.pallas import tpu as pltpu
from jax.experimental.pallas import tpu_sc as plsc

F32 = jnp.float32

_hdot = functools.partial(jnp.dot, precision=lax.Precision.HIGHEST,
                          preferred_element_type=F32)
_fdot = functools.partial(jnp.dot, precision=lax.Precision.DEFAULT,
                          preferred_element_type=F32)


BF16 = jnp.bfloat16


def _split2(v):
    hi = v.astype(BF16)
    lo = (v - hi.astype(F32)).astype(BF16)
    return hi, lo


def _bdot(a, b):
    return jnp.dot(a, b, preferred_element_type=F32)


def _dot3(a1, a2, bhi, blo):
    return (_bdot(a1, bhi) + _bdot(a1, blo)) + _bdot(a2, bhi)


def _route_body(x_ref, lmat_ref, pmat_ref, w1hi_ref, w1lo_ref, b1_ref,
                w2hi_ref, w2lo_ref, b2_ref,
                xg_ref, top2_ref, m_ref, idx2_ref):
    xb = x_ref[0]
    x1, x2 = _split2(xb)
    lm = lmat_ref[...]
    z = _bdot(lm, x1) + _bdot(lm, x2)
    z1, z2 = _split2(z)
    pm = pmat_ref[...]
    pooled = (_bdot(z1, pm) + _bdot(z2, pm)) * (1.0 / 196.0)

    p1, p2 = _split2(pooled)
    accs = [None] * 4
    for a in range(48):
        d = _dot3(p1[a:a + 1, :], p2[a:a + 1, :], w1hi_ref[a], w1lo_ref[a])
        g = a % 4
        accs[g] = d if accs[g] is None else accs[g] + d
    acc = b1_ref[...] + ((accs[0] + accs[1]) + (accs[2] + accs[3]))
    f = jax.nn.relu(acc)
    f1, f2 = _split2(f)
    lg = _dot3(f1, f2, w2hi_ref[...], w2lo_ref[...]) + b2_ref[...]

    io64 = lax.broadcasted_iota(jnp.int32, (1, 64), 1)
    m1 = jnp.max(lg, axis=1, keepdims=True)
    i1 = jnp.min(jnp.where(lg == m1, io64, 64), axis=1, keepdims=True)
    lg2 = jnp.where(io64 == i1, F32(-1e30), lg)
    m2 = jnp.max(lg2, axis=1, keepdims=True)
    i2 = jnp.min(jnp.where(lg2 == m2, io64, 64), axis=1, keepdims=True)
    io128 = lax.broadcasted_iota(jnp.int32, (1, 128), 1)
    top2_ref[0] = jnp.where(io128 == 0,
                            jnp.broadcast_to(i1, (1, 128)),
                            jnp.broadcast_to(i2, (1, 128)))

    iob = lax.broadcasted_iota(jnp.int32, (1, 512), 1)
    e_selb = jnp.where(iob < 256,
                       jnp.broadcast_to(i1, (1, 512)),
                       jnp.broadcast_to(i2, (1, 512)))
    idx2_ref[0] = e_selb * 256 + (iob & 255)

    parts = [f]
    for k in range(4):
        q = f[:, k * 128:(k + 1) * 128]
        parts.append(jnp.pad(q, ((0, 0), (0, 384))))
    xg = jnp.concatenate(parts, axis=0)
    xg_ref[0] = xg

    diff = xg[:, None, :] - xg[None, :, :]
    d2 = jnp.sum(diff * diff, axis=-1)
    io5c = lax.broadcasted_iota(jnp.int32, (5, 5), 1)
    io5r = lax.broadcasted_iota(jnp.int32, (5, 5), 0)
    mx = jnp.max(d2, axis=1, keepdims=True)
    excl = jnp.max(jnp.where(d2 == mx, io5c, -1), axis=1, keepdims=True)
    m_ref[0] = ((io5c != excl).astype(F32) + (io5c == io5r).astype(F32))


def _route(x3, lmat, pmat, w1, b1, w2, b2):
    bn = x3.shape[0]
    w1b = w1.reshape(48, 16, 512)
    w1hi = w1b.astype(BF16)
    w1lo = (w1b - w1hi.astype(F32)).astype(BF16)
    w2hi = w2.astype(BF16)
    w2lo = (w2 - w2hi.astype(F32)).astype(BF16)
    return pl.pallas_call(
        _route_body,
        grid=(bn,),
        in_specs=[
            pl.BlockSpec((1, 672, 224), lambda b: (b, 0, 0)),
            pl.BlockSpec((48, 672), lambda b: (0, 0)),
            pl.BlockSpec((224, 16), lambda b: (0, 0)),
            pl.BlockSpec((48, 16, 512), lambda b: (0, 0, 0)),
            pl.BlockSpec((48, 16, 512), lambda b: (0, 0, 0)),
            pl.BlockSpec((1, 512), lambda b: (0, 0)),
            pl.BlockSpec((512, 64), lambda b: (0, 0)),
            pl.BlockSpec((512, 64), lambda b: (0, 0)),
            pl.BlockSpec((1, 64), lambda b: (0, 0)),
        ],
        out_specs=[
            pl.BlockSpec((1, 5, 512), lambda b: (b, 0, 0)),
            pl.BlockSpec((1, 1, 128), lambda b: (b, 0, 0)),
            pl.BlockSpec((1, 5, 5), lambda b: (b, 0, 0)),
            pl.BlockSpec((1, 1, 512), lambda b: (b, 0, 0)),
        ],
        out_shape=[
            jax.ShapeDtypeStruct((bn, 5, 512), F32),
            jax.ShapeDtypeStruct((bn, 1, 128), jnp.int32),
            jax.ShapeDtypeStruct((bn, 5, 5), F32),
            jax.ShapeDtypeStruct((bn, 1, 512), jnp.int32),
        ],
    )(x3, lmat.astype(BF16), pmat.astype(BF16), w1hi, w1lo,
      b1.reshape(1, 512), w2hi, w2lo, b2.reshape(1, 64))


def _sc_gather(idx2_flat, top2_flat, w2_view, b1, b2):
    mesh = plsc.VectorSubcoreMesh(core_axis_name="c", subcore_axis_name="s")

    @functools.partial(
        pl.kernel,
        out_type=(
            jax.ShapeDtypeStruct((4096, 128), F32),
            jax.ShapeDtypeStruct((16, 256), F32),
            jax.ShapeDtypeStruct((16, 128), F32),
        ),
        mesh=mesh,
        scratch_types=(
            pltpu.VMEM((128,), jnp.int32),
            pltpu.VMEM((16,), jnp.int32),
            pltpu.VMEM((128, 128), F32),
            pltpu.VMEM((16, 256), F32),
            pltpu.VMEM((16, 128), F32),
            pltpu.SemaphoreType.DMA,
            pltpu.SemaphoreType.DMA,
            pltpu.SemaphoreType.DMA,
        ),
    )
    def gather_k(idx2_hbm, top2_hbm, w2_hbm, b1_hbm, b2_hbm,
                 o_w2, o_b1, o_b2,
                 ic_v, e_v, rows2_v, b1_v, b2_v,
                 sem, semb, semo):
        wid = lax.axis_index("s") * 2 + lax.axis_index("c")
        pltpu.sync_copy(idx2_hbm.at[pl.ds(wid * 128, 128)], ic_v)
        c3 = pltpu.async_copy(w2_hbm.at[ic_v], rows2_v, sem)
        c3.wait()
        o3 = pltpu.async_copy(rows2_v, o_w2.at[pl.ds(wid * 128, 128)], semo)
        o3.wait()

        @pl.when(wid == 0)
        def _():
            pltpu.sync_copy(top2_hbm, e_v)
            pltpu.async_copy(b1_hbm.at[e_v], b1_v, semb).wait()
            pltpu.sync_copy(b1_v, o_b1)

        @pl.when(wid == 1)
        def _():
            pltpu.sync_copy(top2_hbm, e_v)
            pltpu.async_copy(b2_hbm.at[e_v], b2_v, semb).wait()
            pltpu.sync_copy(b2_v, o_b2)

    return gather_k(idx2_flat, top2_flat, w2_view, b1, b2)


def _experts_body(e_ref, xg_ref, m_ref, w1a_ref, w1b_ref, w2_ref,
                  b1_ref, b2_ref, fw_ref, fb_ref, o_ref, acc_ref):
    t = pl.program_id(0)
    xg = xg_ref[0]
    mm = m_ref[0]
    for j, w1_ref in ((0, w1a_ref), (1, w1b_ref)):
        p = 2 * t + j
        xw = _fdot(xg, w1_ref[0])
        h = jax.nn.relu(_fdot(mm, xw) * 0.2 + b1_ref[pl.ds(p, 1), :])
        h2 = _fdot(mm, _fdot(h, w2_ref[j])) * 0.2 + b2_ref[pl.ds(p, 1), :]
        acc_ref[pl.ds(p, 1), :] = jnp.mean(h2, axis=0, keepdims=True)

    @pl.when(t == 7)
    def _():
        comb = acc_ref[...]
        fin = _fdot(comb, fw_ref[...])
        io5c = lax.broadcasted_iota(jnp.int32, (5, 5), 1)
        io5r = lax.broadcasted_iota(jnp.int32, (5, 5), 0)
        c7 = mm - (io5c == io5r).astype(F32)
        c7p = jnp.pad(c7, ((0, 11), (0, 11)))
        r16 = lax.broadcasted_iota(jnp.int32, (16, 16), 0)
        c16 = lax.broadcasted_iota(jnp.int32, (16, 16), 1)
        diag = jnp.where(r16 == c16,
                         jnp.where(r16 < 5, F32(0.2), F32(1.0)), F32(0.0))
        mf = diag + c7p * 0.2
        fin2 = _fdot(mf, fin) + fb_ref[...]
        r8 = lax.broadcasted_iota(jnp.int32, (8, 16), 0)
        c8 = lax.broadcasted_iota(jnp.int32, (8, 16), 1)
        pairmean = ((c8 == 2 * r8) | (c8 == 2 * r8 + 1)).astype(F32)
        o_ref[...] = _fdot(pairmean, fin2) * 0.5


def _experts(top2i, xg, m, w1_full, w2g, b1g, b2g, fw, fb):
    grid_spec = pltpu.PrefetchScalarGridSpec(
        num_scalar_prefetch=1,
        grid=(8,),
        in_specs=[
            pl.BlockSpec((1, 5, 512), lambda t, e: (t, 0, 0)),
            pl.BlockSpec((1, 5, 5), lambda t, e: (t, 0, 0)),
            pl.BlockSpec((1, 512, 256), lambda t, e: (e[2 * t], 0, 0)),
            pl.BlockSpec((1, 512, 256), lambda t, e: (e[2 * t + 1], 0, 0)),
            pl.BlockSpec((2, 256, 128), lambda t, e: (t, 0, 0)),
            pl.BlockSpec((16, 256), lambda t, e: (0, 0)),
            pl.BlockSpec((16, 128), lambda t, e: (0, 0)),
            pl.BlockSpec((128, 64), lambda t, e: (0, 0)),
            pl.BlockSpec((1, 64), lambda t, e: (0, 0)),
        ],
        out_specs=pl.BlockSpec((8, 64), lambda t, e: (0, 0)),
        scratch_shapes=[pltpu.VMEM((16, 128), F32)],
    )
    return pl.pallas_call(
        _experts_body,
        grid_spec=grid_spec,
        out_shape=jax.ShapeDtypeStruct((8, 64), F32),
    )(top2i, xg, m, w1_full, w1_full, w2g, b1g, b2g, fw, fb)


def _make_pool_consts():
    lmat = np.zeros((48, 672), dtype=np.float32)
    for a in range(48):
        ch, i = divmod(a, 16)
        lmat[a, ch * 224 + i * 14:(ch * 224 + (i + 1) * 14)] = 1.0
    pmat = np.zeros((224, 16), dtype=np.float32)
    for rr in range(224):
        pmat[rr, rr // 14] = 1.0
    return jnp.asarray(lmat), jnp.asarray(pmat)


def kernel(x, yolo_W1, yolo_b1, yolo_W2, yolo_b2,
           gnn_W1, gnn_b1, gnn_W2, gnn_b2, final_W, final_b):
    bn = x.shape[0]
    lmat, pmat = _make_pool_consts()
    xg, top2_3d, m, idx2_3d = _route(
        x.reshape(bn, 672, 224), lmat, pmat,
        yolo_W1, yolo_b1, yolo_W2, yolo_b2)
    top2_flat = top2_3d[:, 0, :2].reshape(2 * bn).astype(jnp.int32)

    o_w2, b1g, b2g = _sc_gather(
        idx2_3d.reshape(512 * bn), top2_flat,
        gnn_W2.reshape(64 * 256, 128),
        gnn_b1, gnn_b2)
    w2g = o_w2.reshape(16, 256, 128)

    return _experts(top2_flat, xg, m, gnn_W1, w2g, b1g, b2g,
                    final_W, final_b.reshape(1, 64))


# speedup vs baseline: 1.1967x; 1.0440x over previous
import functools

import jax
import jax.numpy as jnp
import numpy as np
from jax import lax
from jax.experimental import pallas as pl
from jax.experimental.pallas import tpu as pltpu
from jax.experimental.pallas import tpu_sc as plsc

F32 = jnp.float32

_hdot = functools.partial(jnp.dot, precision=lax.Precision.HIGHEST,
                          preferred_element_type=F32)
_fdot = functools.partial(jnp.dot, precision=lax.Precision.DEFAULT,
                          preferred_element_type=F32)


BF16 = jnp.bfloat16


def _split2(v):
    hi = v.astype(BF16)
    lo = (v - hi.astype(F32)).astype(BF16)
    return hi, lo


def _bdot(a, b):
    return jnp.dot(a, b, preferred_element_type=F32)


def _dot3(a1, a2, bhi, blo):
    return (_bdot(a1, bhi) + _bdot(a1, blo)) + _bdot(a2, bhi)


def _route_body(x_ref, lmat_ref, pmat_ref, w1hi_ref, w1lo_ref, b1_ref,
                w2hi_ref, w2lo_ref, b2_ref,
                xg_ref, top2_ref, m_ref, idx2_ref):
    xb = x_ref[0]
    x1, x2 = _split2(xb)
    lm = lmat_ref[...]
    z = _bdot(lm, x1) + _bdot(lm, x2)
    z1, z2 = _split2(z)
    pm = pmat_ref[...]
    pooled = (_bdot(z1, pm) + _bdot(z2, pm)) * (1.0 / 196.0)

    p1, p2 = _split2(pooled)
    accs = [None] * 4
    for a in range(48):
        d = _dot3(p1[a:a + 1, :], p2[a:a + 1, :], w1hi_ref[a], w1lo_ref[a])
        g = a % 4
        accs[g] = d if accs[g] is None else accs[g] + d
    acc = b1_ref[...] + ((accs[0] + accs[1]) + (accs[2] + accs[3]))
    f = jax.nn.relu(acc)
    f1, f2 = _split2(f)
    lg = _dot3(f1, f2, w2hi_ref[...], w2lo_ref[...]) + b2_ref[...]

    io64 = lax.broadcasted_iota(jnp.int32, (1, 64), 1)
    m1 = jnp.max(lg, axis=1, keepdims=True)
    i1 = jnp.min(jnp.where(lg == m1, io64, 64), axis=1, keepdims=True)
    lg2 = jnp.where(io64 == i1, F32(-1e30), lg)
    m2 = jnp.max(lg2, axis=1, keepdims=True)
    i2 = jnp.min(jnp.where(lg2 == m2, io64, 64), axis=1, keepdims=True)
    io128 = lax.broadcasted_iota(jnp.int32, (1, 128), 1)
    top2_ref[0] = jnp.where(io128 == 0,
                            jnp.broadcast_to(i1, (1, 128)),
                            jnp.broadcast_to(i2, (1, 128)))

    iob = lax.broadcasted_iota(jnp.int32, (1, 512), 1)
    e_selb = jnp.where(iob < 256,
                       jnp.broadcast_to(i1, (1, 512)),
                       jnp.broadcast_to(i2, (1, 512)))
    idx2_ref[0] = e_selb * 256 + (iob & 255)

    parts = [f]
    for k in range(4):
        q = f[:, k * 128:(k + 1) * 128]
        parts.append(jnp.pad(q, ((0, 0), (0, 384))))
    xg = jnp.concatenate(parts, axis=0)
    xg_ref[0] = xg

    diff = xg[:, None, :] - xg[None, :, :]
    d2 = jnp.sum(diff * diff, axis=-1)
    io5c = lax.broadcasted_iota(jnp.int32, (5, 5), 1)
    io5r = lax.broadcasted_iota(jnp.int32, (5, 5), 0)
    mx = jnp.max(d2, axis=1, keepdims=True)
    excl = jnp.max(jnp.where(d2 == mx, io5c, -1), axis=1, keepdims=True)
    m_ref[0] = ((io5c != excl).astype(F32) + (io5c == io5r).astype(F32))


def _route(x3, lmat, pmat, w1, b1, w2, b2):
    bn = x3.shape[0]
    w1b = w1.reshape(48, 16, 512)
    w1hi = w1b.astype(BF16)
    w1lo = (w1b - w1hi.astype(F32)).astype(BF16)
    w2hi = w2.astype(BF16)
    w2lo = (w2 - w2hi.astype(F32)).astype(BF16)
    return pl.pallas_call(
        _route_body,
        grid=(bn,),
        in_specs=[
            pl.BlockSpec((1, 672, 224), lambda b: (b, 0, 0)),
            pl.BlockSpec((48, 672), lambda b: (0, 0)),
            pl.BlockSpec((224, 16), lambda b: (0, 0)),
            pl.BlockSpec((48, 16, 512), lambda b: (0, 0, 0)),
            pl.BlockSpec((48, 16, 512), lambda b: (0, 0, 0)),
            pl.BlockSpec((1, 512), lambda b: (0, 0)),
            pl.BlockSpec((512, 64), lambda b: (0, 0)),
            pl.BlockSpec((512, 64), lambda b: (0, 0)),
            pl.BlockSpec((1, 64), lambda b: (0, 0)),
        ],
        out_specs=[
            pl.BlockSpec((1, 5, 512), lambda b: (b, 0, 0)),
            pl.BlockSpec((1, 1, 128), lambda b: (b, 0, 0)),
            pl.BlockSpec((1, 5, 5), lambda b: (b, 0, 0)),
            pl.BlockSpec((1, 1, 512), lambda b: (b, 0, 0)),
        ],
        out_shape=[
            jax.ShapeDtypeStruct((bn, 5, 512), F32),
            jax.ShapeDtypeStruct((bn, 1, 128), jnp.int32),
            jax.ShapeDtypeStruct((bn, 5, 5), F32),
            jax.ShapeDtypeStruct((bn, 1, 512), jnp.int32),
        ],
    )(x3, lmat.astype(BF16), pmat.astype(BF16), w1hi, w1lo,
      b1.reshape(1, 512), w2hi, w2lo, b2.reshape(1, 64))


def _sc_gather(idx2_flat, w2_view):
    mesh = plsc.VectorSubcoreMesh(core_axis_name="c", subcore_axis_name="s")

    @functools.partial(
        pl.kernel,
        out_type=jax.ShapeDtypeStruct((4096, 128), F32),
        mesh=mesh,
        scratch_types=(
            pltpu.VMEM((128,), jnp.int32),
            pltpu.VMEM((128, 128), F32),
            pltpu.SemaphoreType.DMA,
            pltpu.SemaphoreType.DMA,
        ),
    )
    def gather_k(idx2_hbm, w2_hbm, o_w2, ic_v, rows2_v, sem, semo):
        wid = lax.axis_index("s") * 2 + lax.axis_index("c")
        pltpu.sync_copy(idx2_hbm.at[pl.ds(wid * 128, 128)], ic_v)
        c3 = pltpu.async_copy(w2_hbm.at[ic_v], rows2_v, sem)
        c3.wait()
        o3 = pltpu.async_copy(rows2_v, o_w2.at[pl.ds(wid * 128, 128)], semo)
        o3.wait()

    return gather_k(idx2_flat, w2_view)


def _experts_body(e_ref, xg_ref, m_ref, w1a_ref, w1b_ref, w2_ref,
                  b1a_ref, b1b_ref, b2a_ref, b2b_ref,
                  fw_ref, fb_ref, o_ref, acc_ref):
    t = pl.program_id(0)
    xg = xg_ref[0]
    mm = m_ref[0]
    for j, w1_ref, b1_ref, b2_ref in ((0, w1a_ref, b1a_ref, b2a_ref),
                                      (1, w1b_ref, b1b_ref, b2b_ref)):
        p = 2 * t + j
        xw = _fdot(xg, w1_ref[0])
        h = jax.nn.relu(_fdot(mm, xw) * 0.2 + b1_ref[0])
        h2 = _fdot(mm, _fdot(h, w2_ref[j])) * 0.2 + b2_ref[0]
        acc_ref[pl.ds(p, 1), :] = jnp.mean(h2, axis=0, keepdims=True)

    @pl.when(t == 7)
    def _():
        comb = acc_ref[...]
        fin = _fdot(comb, fw_ref[...])
        io5c = lax.broadcasted_iota(jnp.int32, (5, 5), 1)
        io5r = lax.broadcasted_iota(jnp.int32, (5, 5), 0)
        c7 = mm - (io5c == io5r).astype(F32)
        c7p = jnp.pad(c7, ((0, 11), (0, 11)))
        r16 = lax.broadcasted_iota(jnp.int32, (16, 16), 0)
        c16 = lax.broadcasted_iota(jnp.int32, (16, 16), 1)
        diag = jnp.where(r16 == c16,
                         jnp.where(r16 < 5, F32(0.2), F32(1.0)), F32(0.0))
        mf = diag + c7p * 0.2
        fin2 = _fdot(mf, fin) + fb_ref[...]
        r8 = lax.broadcasted_iota(jnp.int32, (8, 16), 0)
        c8 = lax.broadcasted_iota(jnp.int32, (8, 16), 1)
        pairmean = ((c8 == 2 * r8) | (c8 == 2 * r8 + 1)).astype(F32)
        o_ref[...] = _fdot(pairmean, fin2) * 0.5


def _experts(top2m, xg, m, w1_full, w2g, b1t, b2t, fw, fb):
    grid_spec = pltpu.PrefetchScalarGridSpec(
        num_scalar_prefetch=1,
        grid=(8,),
        in_specs=[
            pl.BlockSpec((1, 5, 512), lambda t, e: (t, 0, 0)),
            pl.BlockSpec((1, 5, 5), lambda t, e: (t, 0, 0)),
            pl.BlockSpec((1, 512, 256), lambda t, e: (e[t, 0], 0, 0)),
            pl.BlockSpec((1, 512, 256), lambda t, e: (e[t, 1], 0, 0)),
            pl.BlockSpec((2, 256, 128), lambda t, e: (t, 0, 0)),
            pl.BlockSpec((1, 1, 256), lambda t, e: (e[t, 0], 0, 0)),
            pl.BlockSpec((1, 1, 256), lambda t, e: (e[t, 1], 0, 0)),
            pl.BlockSpec((1, 1, 128), lambda t, e: (e[t, 0], 0, 0)),
            pl.BlockSpec((1, 1, 128), lambda t, e: (e[t, 1], 0, 0)),
            pl.BlockSpec((128, 64), lambda t, e: (0, 0)),
            pl.BlockSpec((1, 64), lambda t, e: (0, 0)),
        ],
        out_specs=pl.BlockSpec((8, 64), lambda t, e: (0, 0)),
        scratch_shapes=[pltpu.VMEM((16, 128), F32)],
    )
    return pl.pallas_call(
        _experts_body,
        grid_spec=grid_spec,
        out_shape=jax.ShapeDtypeStruct((8, 64), F32),
    )(top2m, xg, m, w1_full, w1_full, w2g, b1t, b1t, b2t, b2t, fw, fb)


def _make_pool_consts():
    lmat = np.zeros((48, 672), dtype=np.float32)
    for a in range(48):
        ch, i = divmod(a, 16)
        lmat[a, ch * 224 + i * 14:(ch * 224 + (i + 1) * 14)] = 1.0
    pmat = np.zeros((224, 16), dtype=np.float32)
    for rr in range(224):
        pmat[rr, rr // 14] = 1.0
    return jnp.asarray(lmat), jnp.asarray(pmat)


def kernel(x, yolo_W1, yolo_b1, yolo_W2, yolo_b2,
           gnn_W1, gnn_b1, gnn_W2, gnn_b2, final_W, final_b):
    bn = x.shape[0]
    lmat, pmat = _make_pool_consts()
    xg, top2_3d, m, idx2_3d = _route(
        x.reshape(bn, 672, 224), lmat, pmat,
        yolo_W1, yolo_b1, yolo_W2, yolo_b2)

    o_w2 = _sc_gather(idx2_3d.reshape(512 * bn),
                      gnn_W2.reshape(64 * 256, 128))
    w2g = o_w2.reshape(16, 256, 128)

    return _experts(top2_3d.reshape(bn, 128), xg, m, gnn_W1, w2g,
                    gnn_b1.reshape(64, 1, 256), gnn_b2.reshape(64, 1, 128),
                    final_W, final_b.reshape(1, 64))
